# baseline (speedup 1.0000x reference)
import functools

import jax
import jax.numpy as jnp
from jax import lax
from jax.experimental import pallas as pl
from jax.experimental.pallas import tpu as pltpu, tpu_sc as plsc

N = 4096
NN = 2 * N
E = 65536
B = 64
NF = 128
H = 64
F = 3 * H
S = 6 * H
RB = 256
NBLK = N // RB

NC, NS = 2, 16
EPW = (2 * E) // (NC * NS)
ROWS_PW = EPW // 128

_mesh = plsc.VectorSubcoreMesh(
    core_axis_name="c", subcore_axis_name="s", num_cores=NC, num_subcores=NS)

_f32 = jnp.float32
_HI = jax.lax.Precision.DEFAULT
_HX = jax.lax.Precision.HIGHEST


def _zero_vmem(ref, rows, cols):
    z = jnp.zeros((16,), _f32)

    @pl.loop(0, rows)
    def _(e):
        for k in range(cols // 16):
            ref[e, pl.ds(16 * k, 16)] = z


@functools.partial(
    pl.kernel, mesh=_mesh,
    out_type=jax.ShapeDtypeStruct((NN,), _f32),
    compiler_params=pltpu.CompilerParams(
        needs_layout_passes=False, use_tc_tiling_on_sc=False),
    scratch_types=[
        pltpu.VMEM((ROWS_PW, 128), jnp.int32),
        pltpu.VMEM((ROWS_PW, 128), _f32),
        pltpu.VMEM((256,), _f32),
        pltpu.VMEM_SHARED((N,), _f32),
    ],
)
def _sc_deg(dst2d, w2d, degp, dst_i, w_i, zb, acc):
    c = lax.axis_index("c")
    s = lax.axis_index("s")
    base = c * 512 + s * ROWS_PW
    pltpu.sync_copy(dst2d.at[pl.ds(base, ROWS_PW)], dst_i)
    pltpu.sync_copy(w2d.at[pl.ds(base, ROWS_PW)], w_i)
    z = jnp.zeros((16,), _f32)

    @pl.loop(0, 16)
    def _(e):
        zb[pl.ds(e * 16, 16)] = z

    pltpu.sync_copy(zb, acc.at[pl.ds(s * 256, 256)])
    plsc.subcore_barrier()
    for j in range(ROWS_PW):
        pltpu.sync_copy(w_i.at[j], acc.at[dst_i.at[j]], add=True)
    plsc.subcore_barrier()
    pltpu.sync_copy(acc.at[pl.ds(s * 256, 256)], zb)
    pltpu.sync_copy(zb, degp.at[pl.ds(c * N + s * 256, 256)])


@functools.partial(
    pl.kernel, mesh=_mesh,
    out_type=jax.ShapeDtypeStruct((2 * E // 128, 128), _f32),
    compiler_params=pltpu.CompilerParams(
        needs_layout_passes=False, use_tc_tiling_on_sc=False),
    scratch_types=[
        pltpu.VMEM((NN // 128, 128), _f32),
        pltpu.VMEM((ROWS_PW, 128), jnp.int32),
        pltpu.VMEM((ROWS_PW, 128), jnp.int32),
        pltpu.VMEM((ROWS_PW, 128), _f32),
        pltpu.VMEM((ROWS_PW, 128), _f32),
    ],
)
def _sc_norm(dinv2d, src2d, dst2d, w2d, norm2d, dv, src_i, dst_i, w_i, nrm):
    c = lax.axis_index("c")
    s = lax.axis_index("s")
    base = c * 512 + s * ROWS_PW
    pltpu.sync_copy(dinv2d, dv)
    pltpu.sync_copy(src2d.at[pl.ds(base, ROWS_PW)], src_i)
    pltpu.sync_copy(dst2d.at[pl.ds(base, ROWS_PW)], dst_i)
    pltpu.sync_copy(w2d.at[pl.ds(base, ROWS_PW)], w_i)
    off = c * N

    @pl.loop(0, ROWS_PW)
    def _(j):
        for k in range(8):
            sl = pl.ds(16 * k, 16)
            sv = src_i[j, sl]
            dvv = dst_i[j, sl] + off
            a = plsc.load_gather(dv, [sv >> 7, sv & 127])
            b_ = plsc.load_gather(dv, [dvv >> 7, dvv & 127])
            nrm[j, sl] = a * b_ * w_i[j, sl]

    pltpu.sync_copy(nrm, norm2d.at[pl.ds(base, ROWS_PW)])


@functools.partial(
    pl.kernel, mesh=_mesh,
    out_type=jax.ShapeDtypeStruct((NN, H), _f32),
    compiler_params=pltpu.CompilerParams(
        needs_layout_passes=False, use_tc_tiling_on_sc=False),
    scratch_types=[
        pltpu.VMEM((ROWS_PW, 128), jnp.int32),
        pltpu.VMEM((ROWS_PW, 128), jnp.int32),
        pltpu.VMEM((ROWS_PW, 128), _f32),
        pltpu.VMEM((4, 128, H), _f32),
        pltpu.VMEM((4, 128, H), _f32),
        pltpu.VMEM((128, H), _f32),
        pltpu.VMEM_SHARED((N, H), _f32),
        pltpu.SemaphoreType.DMA,
        pltpu.SemaphoreType.DMA,
        pltpu.SemaphoreType.DMA,
        pltpu.SemaphoreType.DMA,
        pltpu.SemaphoreType.DMA,
        pltpu.SemaphoreType.DMA,
        pltpu.SemaphoreType.DMA,
        pltpu.SemaphoreType.DMA,
    ],
)
def _sc_prop(xw_h, src2d, dst2d, norm2d, agg, src_i, dst_i, nrm, gbuf, sbuf,
             zb, acc, g0, g1, g2, g3, s0, s1, s2, s3):
    c = lax.axis_index("c")
    s = lax.axis_index("s")
    base = c * 512 + s * ROWS_PW
    pltpu.sync_copy(src2d.at[pl.ds(base, ROWS_PW)], src_i)
    pltpu.sync_copy(dst2d.at[pl.ds(base, ROWS_PW)], dst_i)
    pltpu.sync_copy(norm2d.at[pl.ds(base, ROWS_PW)], nrm)
    _zero_vmem(zb, 128, H)
    pltpu.sync_copy(zb, acc.at[pl.ds(s * 256, 128)])
    pltpu.sync_copy(zb, acc.at[pl.ds(s * 256 + 128, 128)])
    plsc.subcore_barrier()

    gsem = (g0, g1, g2, g3)
    ssem = (s0, s1, s2, s3)

    def gstart(j, b):
        pltpu.async_copy(xw_h.at[src_i.at[j]], gbuf.at[b], gsem[b])

    def gwait(j, b):
        pltpu.make_async_copy(xw_h.at[src_i.at[j]], gbuf.at[b], gsem[b]).wait()

    def sstart(j, b):
        pltpu.async_copy(sbuf.at[b], acc.at[dst_i.at[j]], ssem[b], add=True)

    def swait(j, b):
        pltpu.make_async_copy(sbuf.at[b], acc.at[dst_i.at[j]], ssem[b]).wait()

    def scale(jj, b):
        @pl.loop(0, 8)
        def _(m):
            nvec = nrm[jj, pl.ds(m * 16, 16)]
            for l in range(16):
                nv = nvec[l]
                for k in range(H // 16):
                    sl = pl.ds(16 * k, 16)
                    sbuf[b, m * 16 + l, sl] = gbuf[b, m * 16 + l, sl] * nv

    for b in range(4):
        gstart(b, b)
    for b in range(4):
        gwait(b, b)
        scale(b, b)
        sstart(b, b)
        gstart(b + 4, b)

    @pl.loop(1, 7)
    def _(g):
        for b in range(4):
            j = 4 * g + b
            gwait(j, b)
            swait(j - 4, b)
            scale(j, b)
            sstart(j, b)
            gstart(j + 4, b)

    for b in range(4):
        j = 28 + b
        gwait(j, b)
        swait(j - 4, b)
        scale(j, b)
        sstart(j, b)
    for b in range(4):
        swait(28 + b, b)

    plsc.subcore_barrier()
    for t in range(2):
        pltpu.sync_copy(acc.at[pl.ds(s * 256 + t * 128, 128)], zb)
        pltpu.sync_copy(zb, agg.at[pl.ds(c * N + s * 256 + t * 128, 128)])


def _lrelu(t):
    return jnp.where(t >= 0, t, 0.2 * t)


def _tc_pre_body(degp, x, w0, dinv, xw0):
    deg = 1.0 + degp[...]
    dinv[...] = lax.rsqrt(jnp.maximum(deg, 1e-12))
    xw0[...] = jnp.dot(x[...], w0[...], preferred_element_type=_f32,
                       precision=_HI)


def _tc_pre(degp, x, w0):
    return pl.pallas_call(
        _tc_pre_body,
        out_shape=(jax.ShapeDtypeStruct((NN, 1), _f32),
                   jax.ShapeDtypeStruct((NN, H), _f32)),
    )(degp, x, w0)


def _tc_fuse_body(agg, xw, dinv, b, wn, fc, xwn):
    d2 = dinv[...] * dinv[...]
    f = _lrelu(agg[...] + d2 * xw[...] + b[...])
    fc[...] = f
    xwn[...] = jnp.dot(f, wn[...], preferred_element_type=_f32, precision=_HI)


def _tc_fuse(agg, xw, dinv, b, wn):
    return pl.pallas_call(
        _tc_fuse_body,
        out_shape=(jax.ShapeDtypeStruct((NN, H), _f32),
                   jax.ShapeDtypeStruct((NN, H), _f32)),
    )(agg, xw, dinv, b.reshape(1, H), wn)


def _set2set(x, P, PT, Wih, Whh, bih, bhh):
    q_star = jnp.zeros((B, 2 * S), _f32)
    h = jnp.zeros((B, S), _f32)
    cc = jnp.zeros((B, S), _f32)
    dn = lambda a, b_, dims: lax.dot_general(
        a, b_, (dims, ((), ())), preferred_element_type=_f32, precision=_HI)
    for _ in range(2):
        g = (dn(q_star, Wih, ((1,), (1,))) + bih
             + dn(h, Whh, ((1,), (1,))) + bhh)
        gi = g[:, 0:S]
        gf = g[:, S:2 * S]
        gg = g[:, 2 * S:3 * S]
        go = g[:, 3 * S:4 * S]
        sig = lambda t: 1.0 / (1.0 + jnp.exp(-t))
        cc = sig(gf) * cc + sig(gi) * jnp.tanh(gg)
        h = sig(go) * jnp.tanh(cc)
        q = h
        e_all = dn(x, q, ((1,), (1,)))
        e = jnp.sum(e_all * PT, axis=1, keepdims=True)
        em = jnp.max(jnp.where(PT > 0, e_all, -1e30),
                     axis=0, keepdims=True)
        eg = dn(PT, em, ((1,), (1,)))
        ee = jnp.exp(e - eg)
        den = dn(P, ee, ((1,), (0,)))
        dg = dn(P, den, ((0,), (0,)))
        a = ee / (dg + 1e-16)
        pa = P * a[:, 0][None, :]
        r = dn(pa, x, ((1,), (0,)))
        q_star = jnp.concatenate([q, r], axis=1)
    return q_star


def _tc_inter_body(agg2, xw2, dinv, b2, fc0, fc1, b1col, b2row, act,
                   f1f, f2f):
    d2 = dinv[...] * dinv[...]
    fc2 = _lrelu(agg2[...] + d2 * xw2[...] + b2[...])
    f = jnp.concatenate([fc0[...], fc1[...], fc2], axis=1)
    nrm = jnp.sqrt(jnp.sum(f * f, axis=1, keepdims=True))
    f = f / jnp.maximum(nrm, 1e-12)
    f1f[:, 0:F] = f[:N]
    f2f[:, 0:F] = f[N:]
    f1f[:, F:2 * F] = jnp.zeros((N, F), _f32)
    f2f[:, F:2 * F] = jnp.zeros((N, F), _f32)

    dn = lambda a, b_, dims: lax.dot_general(
        a, b_, (dims, ((), ())), preferred_element_type=_f32, precision=_HI)

    def pair(t, _):
        I = t // NBLK
        J = t % NBLK

        @pl.when(act[I, J] > 0)
        def _():
            a1 = f1f[pl.ds(I * RB, RB), 0:F]
            a2 = f2f[pl.ds(J * RB, RB), 0:F]
            cmat = dn(a1, a2, ((1,), (1,)))
            m1 = b1col[pl.ds(I * RB, RB), :]
            m2 = b2row[:, pl.ds(J * RB, RB)]
            cm = jnp.where(m1 == m2, cmat, 0.0)
            f1f[pl.ds(I * RB, RB), F:2 * F] += dn(cm, a2, ((1,), (0,)))
            f2f[pl.ds(J * RB, RB), F:2 * F] += dn(cm, a1, ((0,), (0,)))

        return 0

    lax.fori_loop(0, NBLK * NBLK, pair, 0)


def _tc_inter(agg2, xw2, dinv, b2, fc0, fc1, b1col, b2row, act):
    return pl.pallas_call(
        _tc_inter_body,
        out_shape=(jax.ShapeDtypeStruct((N, 2 * F), _f32),
                   jax.ShapeDtypeStruct((N, 2 * F), _f32)),
        in_specs=[pl.BlockSpec(memory_space=pltpu.VMEM)] * 8
        + [pl.BlockSpec(memory_space=pltpu.SMEM)],
    )(agg2, xw2, dinv, b2.reshape(1, H), fc0, fc1, b1col, b2row, act)


def _tc_s2s_body(xff, P, PT, Wih, Whh, bih, bhh, g_out):
    g_out[...] = _set2set(xff[...], P[...], PT[...], Wih[...], Whh[...],
                          bih[...], bhh[...])


def _tc_s2s(xff, P, PT, Wih, Whh, bih, bhh):
    return pl.pallas_call(
        _tc_s2s_body,
        out_shape=jax.ShapeDtypeStruct((B, 2 * S), _f32),
    )(xff, P, PT, Wih, Whh, bih.reshape(1, 4 * S), bhh.reshape(1, 4 * S))


def _tc_mlp_body(g1, g2, Pw0, Pb0, Pw1, Pb1, Pw2, Pb2, out):
    dn = lambda a, b_, dims: lax.dot_general(
        a, b_, (dims, ((), ())), preferred_element_type=_f32, precision=_HX)
    ff = jnp.concatenate([g1[...], g2[...]], axis=1)
    hdd = jnp.maximum(dn(ff, Pw0[...], ((1,), (0,))) + Pb0[...], 0.0)
    hdd = jnp.maximum(dn(hdd, Pw1[...], ((1,), (0,))) + Pb1[...], 0.0)
    sc = dn(hdd, Pw2[...], ((1,), (0,))) + Pb2[...]
    out[...] = (1.0 / (1.0 + jnp.exp(-sc)))[:, 0]


def _tc_mlp(g1, g2, Pw0, Pb0, Pw1, Pb1, Pw2, Pb2):
    return pl.pallas_call(
        _tc_mlp_body,
        out_shape=jax.ShapeDtypeStruct((B,), _f32),
    )(g1, g2, Pw0, Pb0.reshape(1, 256), Pw1, Pb1.reshape(1, 128),
      Pw2, Pb2.reshape(1, 1))


def kernel(x1, x2, edge_index1, edge_index2, edge_attr1, edge_attr2,
           batch1, batch2, int_map0, int_map1, W0, b0, W1, b1, W2, b2,
           Wih, Whh, bih, bhh, Pw0, Pb0, Pw1, Pb1, Pw2, Pb2):
    ei1 = edge_index1.astype(jnp.int32)
    ei2 = edge_index2.astype(jnp.int32)
    src2d = jnp.concatenate([ei1[0], ei2[0] + N]).reshape(1024, 128)
    dst2d = jnp.concatenate([ei1[1], ei2[1]]).reshape(1024, 128)
    w2d = jnp.concatenate([edge_attr1, edge_attr2]).reshape(1024, 128)
    x = jnp.concatenate([x1, x2], axis=0)

    b1i = batch1.astype(jnp.int32)
    b2i = batch2.astype(jnp.int32)
    blo1, bhi1 = b1i[::RB], b1i[RB - 1::RB]
    blo2, bhi2 = b2i[::RB], b2i[RB - 1::RB]
    act = ((blo1[:, None] <= bhi2[None, :])
           & (blo2[None, :] <= bhi1[:, None])).astype(jnp.int32)
    b1col = batch1.astype(_f32).reshape(N, 1)
    b2row = batch2.astype(_f32).reshape(1, N)

    degp = _sc_deg(dst2d, w2d)
    dinv, xw0 = _tc_pre(degp.reshape(NN, 1), x, W0)
    norm2d = _sc_norm(dinv.reshape(NN // 128, 128), src2d, dst2d, w2d)
    agg0 = _sc_prop(xw0, src2d, dst2d, norm2d)
    fc0, xw1 = _tc_fuse(agg0, xw0, dinv, b0, W1)
    agg1 = _sc_prop(xw1, src2d, dst2d, norm2d)
    fc1, xw2 = _tc_fuse(agg1, xw1, dinv, b1, W2)
    agg2 = _sc_prop(xw2, src2d, dst2d, norm2d)
    f1f, f2f = _tc_inter(agg2, xw2, dinv, b2, fc0, fc1, b1col, b2row, act)
    g1 = _tc_s2s(f1f, int_map0, int_map0.T, Wih, Whh, bih, bhh)
    g2 = _tc_s2s(f2f, int_map1, int_map1.T, Wih, Whh, bih, bhh)
    return _tc_mlp(g1, g2, Pw0, Pb0, Pw1, Pb1, Pw2, Pb2)


# speedup vs baseline: 1.0061x; 1.0061x over previous
import functools

import jax
import jax.numpy as jnp
from jax import lax
from jax.experimental import pallas as pl
from jax.experimental.pallas import tpu as pltpu, tpu_sc as plsc

N = 4096
NN = 2 * N
E = 65536
B = 64
NF = 128
H = 64
F = 3 * H
S = 6 * H
RB = 256
NBLK = N // RB

NC, NS = 2, 16
EPW = (2 * E) // (NC * NS)
ROWS_PW = EPW // 128

_mesh = plsc.VectorSubcoreMesh(
    core_axis_name="c", subcore_axis_name="s", num_cores=NC, num_subcores=NS)

_f32 = jnp.float32
_HI = jax.lax.Precision.DEFAULT
_HX = jax.lax.Precision.HIGHEST


def _zero_vmem(ref, rows, cols):
    z = jnp.zeros((16,), _f32)

    @pl.loop(0, rows)
    def _(e):
        for k in range(cols // 16):
            ref[e, pl.ds(16 * k, 16)] = z


@functools.partial(
    pl.kernel, mesh=_mesh,
    out_type=jax.ShapeDtypeStruct((NN,), _f32),
    compiler_params=pltpu.CompilerParams(
        needs_layout_passes=False, use_tc_tiling_on_sc=False),
    scratch_types=[
        pltpu.VMEM((ROWS_PW, 128), jnp.int32),
        pltpu.VMEM((ROWS_PW, 128), _f32),
        pltpu.VMEM((256,), _f32),
        pltpu.VMEM_SHARED((N,), _f32),
    ],
)
def _sc_deg(dst2d, w2d, degp, dst_i, w_i, zb, acc):
    c = lax.axis_index("c")
    s = lax.axis_index("s")
    base = c * 512 + s * ROWS_PW
    pltpu.sync_copy(dst2d.at[pl.ds(base, ROWS_PW)], dst_i)
    pltpu.sync_copy(w2d.at[pl.ds(base, ROWS_PW)], w_i)
    z = jnp.zeros((16,), _f32)

    @pl.loop(0, 16)
    def _(e):
        zb[pl.ds(e * 16, 16)] = z

    pltpu.sync_copy(zb, acc.at[pl.ds(s * 256, 256)])
    plsc.subcore_barrier()
    for j in range(ROWS_PW):
        pltpu.sync_copy(w_i.at[j], acc.at[dst_i.at[j]], add=True)
    plsc.subcore_barrier()
    pltpu.sync_copy(acc.at[pl.ds(s * 256, 256)], zb)
    pltpu.sync_copy(zb, degp.at[pl.ds(c * N + s * 256, 256)])


@functools.partial(
    pl.kernel, mesh=_mesh,
    out_type=jax.ShapeDtypeStruct((2 * E // 128, 128), _f32),
    compiler_params=pltpu.CompilerParams(
        needs_layout_passes=False, use_tc_tiling_on_sc=False),
    scratch_types=[
        pltpu.VMEM((NN // 128, 128), _f32),
        pltpu.VMEM((ROWS_PW, 128), jnp.int32),
        pltpu.VMEM((ROWS_PW, 128), jnp.int32),
        pltpu.VMEM((ROWS_PW, 128), _f32),
        pltpu.VMEM((ROWS_PW, 128), _f32),
    ],
)
def _sc_norm(dinv2d, src2d, dst2d, w2d, norm2d, dv, src_i, dst_i, w_i, nrm):
    c = lax.axis_index("c")
    s = lax.axis_index("s")
    base = c * 512 + s * ROWS_PW
    pltpu.sync_copy(dinv2d, dv)
    pltpu.sync_copy(src2d.at[pl.ds(base, ROWS_PW)], src_i)
    pltpu.sync_copy(dst2d.at[pl.ds(base, ROWS_PW)], dst_i)
    pltpu.sync_copy(w2d.at[pl.ds(base, ROWS_PW)], w_i)
    off = c * N

    @pl.loop(0, ROWS_PW)
    def _(j):
        for k in range(8):
            sl = pl.ds(16 * k, 16)
            sv = src_i[j, sl]
            dvv = dst_i[j, sl] + off
            a = plsc.load_gather(dv, [sv >> 7, sv & 127])
            b_ = plsc.load_gather(dv, [dvv >> 7, dvv & 127])
            nrm[j, sl] = a * b_ * w_i[j, sl]

    pltpu.sync_copy(nrm, norm2d.at[pl.ds(base, ROWS_PW)])


@functools.partial(
    pl.kernel, mesh=_mesh,
    out_type=jax.ShapeDtypeStruct((NN, H), _f32),
    compiler_params=pltpu.CompilerParams(
        needs_layout_passes=False, use_tc_tiling_on_sc=False),
    scratch_types=[
        pltpu.VMEM((ROWS_PW, 128), jnp.int32),
        pltpu.VMEM((ROWS_PW, 128), jnp.int32),
        pltpu.VMEM((ROWS_PW, 128), _f32),
        pltpu.VMEM((4, 128, H), _f32),
        pltpu.VMEM((4, 128, H), _f32),
        pltpu.VMEM((128, H), _f32),
        pltpu.VMEM_SHARED((N, H), _f32),
        pltpu.SemaphoreType.DMA,
        pltpu.SemaphoreType.DMA,
        pltpu.SemaphoreType.DMA,
        pltpu.SemaphoreType.DMA,
        pltpu.SemaphoreType.DMA,
        pltpu.SemaphoreType.DMA,
        pltpu.SemaphoreType.DMA,
        pltpu.SemaphoreType.DMA,
    ],
)
def _sc_prop(xw_h, src2d, dst2d, norm2d, agg, src_i, dst_i, nrm, gbuf, sbuf,
             zb, acc, g0, g1, g2, g3, s0, s1, s2, s3):
    c = lax.axis_index("c")
    s = lax.axis_index("s")
    base = c * 512 + s * ROWS_PW
    pltpu.sync_copy(src2d.at[pl.ds(base, ROWS_PW)], src_i)
    pltpu.sync_copy(dst2d.at[pl.ds(base, ROWS_PW)], dst_i)
    pltpu.sync_copy(norm2d.at[pl.ds(base, ROWS_PW)], nrm)
    _zero_vmem(zb, 128, H)
    pltpu.sync_copy(zb, acc.at[pl.ds(s * 256, 128)])
    pltpu.sync_copy(zb, acc.at[pl.ds(s * 256 + 128, 128)])
    plsc.subcore_barrier()

    gsem = (g0, g1, g2, g3)
    ssem = (s0, s1, s2, s3)

    def gstart(j, b):
        pltpu.async_copy(xw_h.at[src_i.at[j]], gbuf.at[b], gsem[b])

    def gwait(j, b):
        pltpu.make_async_copy(xw_h.at[src_i.at[j]], gbuf.at[b], gsem[b]).wait()

    def sstart(j, b):
        pltpu.async_copy(sbuf.at[b], acc.at[dst_i.at[j]], ssem[b], add=True)

    def swait(j, b):
        pltpu.make_async_copy(sbuf.at[b], acc.at[dst_i.at[j]], ssem[b]).wait()

    def scale(jj, b):
        @pl.loop(0, 8)
        def _(m):
            nvec = nrm[jj, pl.ds(m * 16, 16)]
            for l in range(16):
                nv = nvec[l]
                for k in range(H // 16):
                    sl = pl.ds(16 * k, 16)
                    sbuf[b, m * 16 + l, sl] = gbuf[b, m * 16 + l, sl] * nv

    for b in range(4):
        gstart(b, b)
    for b in range(4):
        gwait(b, b)
        scale(b, b)
        sstart(b, b)
        gstart(b + 4, b)

    @pl.loop(1, 7)
    def _(g):
        for b in range(4):
            j = 4 * g + b
            gwait(j, b)
            swait(j - 4, b)
            scale(j, b)
            sstart(j, b)
            gstart(j + 4, b)

    for b in range(4):
        j = 28 + b
        gwait(j, b)
        swait(j - 4, b)
        scale(j, b)
        sstart(j, b)
    for b in range(4):
        swait(28 + b, b)

    plsc.subcore_barrier()
    for t in range(2):
        pltpu.sync_copy(acc.at[pl.ds(s * 256 + t * 128, 128)], zb)
        pltpu.sync_copy(zb, agg.at[pl.ds(c * N + s * 256 + t * 128, 128)])


def _lrelu(t):
    return jnp.where(t >= 0, t, 0.2 * t)


def _tc_pre_body(degp, x, w0, dinv, xw0):
    deg = 1.0 + degp[...]
    dinv[...] = lax.rsqrt(jnp.maximum(deg, 1e-12))
    xw0[...] = jnp.dot(x[...], w0[...], preferred_element_type=_f32,
                       precision=_HI)


def _tc_pre(degp, x, w0):
    return pl.pallas_call(
        _tc_pre_body,
        out_shape=(jax.ShapeDtypeStruct((NN, 1), _f32),
                   jax.ShapeDtypeStruct((NN, H), _f32)),
    )(degp, x, w0)


def _tc_fuse_body(agg, xw, dinv, b, wn, fc, xwn):
    d2 = dinv[...] * dinv[...]
    f = _lrelu(agg[...] + d2 * xw[...] + b[...])
    fc[...] = f
    xwn[...] = jnp.dot(f, wn[...], preferred_element_type=_f32, precision=_HI)


def _tc_fuse(agg, xw, dinv, b, wn):
    return pl.pallas_call(
        _tc_fuse_body,
        out_shape=(jax.ShapeDtypeStruct((NN, H), _f32),
                   jax.ShapeDtypeStruct((NN, H), _f32)),
    )(agg, xw, dinv, b.reshape(1, H), wn)


def _set2set(x, P, Wih, Whh, bih, bhh):
    q_star = jnp.zeros((B, 2 * S), _f32)
    h = jnp.zeros((B, S), _f32)
    cc = jnp.zeros((B, S), _f32)
    dn = lambda a, b_, dims: lax.dot_general(
        a, b_, (dims, ((), ())), preferred_element_type=_f32, precision=_HI)
    for _ in range(2):
        g = (dn(q_star, Wih, ((1,), (1,))) + bih
             + dn(h, Whh, ((1,), (1,))) + bhh)
        gi = g[:, 0:S]
        gf = g[:, S:2 * S]
        gg = g[:, 2 * S:3 * S]
        go = g[:, 3 * S:4 * S]
        sig = lambda t: 1.0 / (1.0 + jnp.exp(-t))
        cc = sig(gf) * cc + sig(gi) * jnp.tanh(gg)
        h = sig(go) * jnp.tanh(cc)
        q = h
        qb = dn(P, q, ((0,), (0,)))
        e = jnp.sum(x * qb, axis=1, keepdims=True)
        em = jnp.max(jnp.where(P > 0, e[:, 0][None, :], -1e30),
                     axis=1, keepdims=True)
        eg = dn(P, em, ((0,), (0,)))
        ee = jnp.exp(e - eg)
        den = dn(P, ee, ((1,), (0,)))
        dg = dn(P, den, ((0,), (0,)))
        a = ee / (dg + 1e-16)
        r = dn(P, a * x, ((1,), (0,)))
        q_star = jnp.concatenate([q, r], axis=1)
    return q_star


def _tc_inter_body(agg2, xw2, dinv, b2, fc0, fc1, b1col, b2row, act,
                   f1f, f2f):
    d2 = dinv[...] * dinv[...]
    fc2 = _lrelu(agg2[...] + d2 * xw2[...] + b2[...])
    f = jnp.concatenate([fc0[...], fc1[...], fc2], axis=1)
    nrm = jnp.sqrt(jnp.sum(f * f, axis=1, keepdims=True))
    f = f / jnp.maximum(nrm, 1e-12)
    f1f[:, 0:F] = f[:N]
    f2f[:, 0:F] = f[N:]
    f1f[:, F:2 * F] = jnp.zeros((N, F), _f32)
    f2f[:, F:2 * F] = jnp.zeros((N, F), _f32)

    dn = lambda a, b_, dims: lax.dot_general(
        a, b_, (dims, ((), ())), preferred_element_type=_f32, precision=_HI)

    def pair(t, _):
        I = t // NBLK
        J = t % NBLK

        @pl.when(act[I, J] > 0)
        def _():
            a1 = f1f[pl.ds(I * RB, RB), 0:F]
            a2 = f2f[pl.ds(J * RB, RB), 0:F]
            cmat = dn(a1, a2, ((1,), (1,)))
            m1 = b1col[pl.ds(I * RB, RB), :]
            m2 = b2row[:, pl.ds(J * RB, RB)]
            cm = jnp.where(m1 == m2, cmat, 0.0)
            f1f[pl.ds(I * RB, RB), F:2 * F] += dn(cm, a2, ((1,), (0,)))
            f2f[pl.ds(J * RB, RB), F:2 * F] += dn(cm, a1, ((0,), (0,)))

        return 0

    lax.fori_loop(0, NBLK * NBLK, pair, 0)


def _tc_inter(agg2, xw2, dinv, b2, fc0, fc1, b1col, b2row, act):
    return pl.pallas_call(
        _tc_inter_body,
        out_shape=(jax.ShapeDtypeStruct((N, 2 * F), _f32),
                   jax.ShapeDtypeStruct((N, 2 * F), _f32)),
        in_specs=[pl.BlockSpec(memory_space=pltpu.VMEM)] * 8
        + [pl.BlockSpec(memory_space=pltpu.SMEM)],
    )(agg2, xw2, dinv, b2.reshape(1, H), fc0, fc1, b1col, b2row, act)


def _tc_s2s_body(xff, P, Wih, Whh, bih, bhh, g_out):
    g_out[...] = _set2set(xff[...], P[...], Wih[...], Whh[...],
                          bih[...], bhh[...])


def _tc_s2s(xff, P, Wih, Whh, bih, bhh):
    return pl.pallas_call(
        _tc_s2s_body,
        out_shape=jax.ShapeDtypeStruct((B, 2 * S), _f32),
    )(xff, P, Wih, Whh, bih.reshape(1, 4 * S), bhh.reshape(1, 4 * S))


def _tc_mlp_body(g1, g2, Pw0, Pb0, Pw1, Pb1, Pw2, Pb2, out):
    dn = lambda a, b_, dims: lax.dot_general(
        a, b_, (dims, ((), ())), preferred_element_type=_f32, precision=_HX)
    ff = jnp.concatenate([g1[...], g2[...]], axis=1)
    hdd = jnp.maximum(dn(ff, Pw0[...], ((1,), (0,))) + Pb0[...], 0.0)
    hdd = jnp.maximum(dn(hdd, Pw1[...], ((1,), (0,))) + Pb1[...], 0.0)
    sc = dn(hdd, Pw2[...], ((1,), (0,))) + Pb2[...]
    out[...] = (1.0 / (1.0 + jnp.exp(-sc)))[:, 0]


def _tc_mlp(g1, g2, Pw0, Pb0, Pw1, Pb1, Pw2, Pb2):
    return pl.pallas_call(
        _tc_mlp_body,
        out_shape=jax.ShapeDtypeStruct((B,), _f32),
    )(g1, g2, Pw0, Pb0.reshape(1, 256), Pw1, Pb1.reshape(1, 128),
      Pw2, Pb2.reshape(1, 1))


def kernel(x1, x2, edge_index1, edge_index2, edge_attr1, edge_attr2,
           batch1, batch2, int_map0, int_map1, W0, b0, W1, b1, W2, b2,
           Wih, Whh, bih, bhh, Pw0, Pb0, Pw1, Pb1, Pw2, Pb2):
    ei1 = edge_index1.astype(jnp.int32)
    ei2 = edge_index2.astype(jnp.int32)
    src2d = jnp.concatenate([ei1[0], ei2[0] + N]).reshape(1024, 128)
    dst2d = jnp.concatenate([ei1[1], ei2[1]]).reshape(1024, 128)
    w2d = jnp.concatenate([edge_attr1, edge_attr2]).reshape(1024, 128)
    x = jnp.concatenate([x1, x2], axis=0)

    b1i = batch1.astype(jnp.int32)
    b2i = batch2.astype(jnp.int32)
    blo1, bhi1 = b1i[::RB], b1i[RB - 1::RB]
    blo2, bhi2 = b2i[::RB], b2i[RB - 1::RB]
    act = ((blo1[:, None] <= bhi2[None, :])
           & (blo2[None, :] <= bhi1[:, None])).astype(jnp.int32)
    b1col = batch1.astype(_f32).reshape(N, 1)
    b2row = batch2.astype(_f32).reshape(1, N)

    degp = _sc_deg(dst2d, w2d)
    dinv, xw0 = _tc_pre(degp.reshape(NN, 1), x, W0)
    norm2d = _sc_norm(dinv.reshape(NN // 128, 128), src2d, dst2d, w2d)
    agg0 = _sc_prop(xw0, src2d, dst2d, norm2d)
    fc0, xw1 = _tc_fuse(agg0, xw0, dinv, b0, W1)
    agg1 = _sc_prop(xw1, src2d, dst2d, norm2d)
    fc1, xw2 = _tc_fuse(agg1, xw1, dinv, b1, W2)
    agg2 = _sc_prop(xw2, src2d, dst2d, norm2d)
    f1f, f2f = _tc_inter(agg2, xw2, dinv, b2, fc0, fc1, b1col, b2row, act)
    g1 = _tc_s2s(f1f, int_map0, Wih, Whh, bih, bhh)
    g2 = _tc_s2s(f2f, int_map1, Wih, Whh, bih, bhh)
    return _tc_mlp(g1, g2, Pw0, Pb0, Pw1, Pb1, Pw2, Pb2)
